# baseline (speedup 1.0000x reference)
import jax
import jax.numpy as jnp
from jax.experimental import pallas as pl


def _mlp_kernel(s_ref, w1_ref, b1_ref, w2_ref, b2_ref, o_ref):
    H = b1_ref.shape[0]
    Bn, n, _ = s_ref.shape
    W1a = w1_ref[:H, :]
    W1b = w1_ref[H:, :]
    w2c = w2_ref[...].astype(jnp.bfloat16)
    b2v = b2_ref[0, 0]
    zero = jnp.zeros((), jnp.bfloat16)

    for b in range(Bn):
        s = s_ref[b]
        AT = jax.lax.dot_general(W1a, s, (((0,), (1,)), ((), ())),
                                 preferred_element_type=jnp.float32) + b1_ref[...]
        BT = jax.lax.dot_general(W1b, s, (((0,), (1,)), ((), ())),
                                 preferred_element_type=jnp.float32)
        ATb = AT.astype(jnp.bfloat16)
        BTb = BT.astype(jnp.bfloat16)
        maskf = (s != 0).astype(jnp.float32)
        for i in range(n):
            col = ATb[:, i:i + 1]
            m = jnp.maximum(BTb + col, zero)
            row = jax.lax.dot_general(w2c, m, (((0,), (0,)), ((), ())),
                                      preferred_element_type=jnp.float32)
            row = jax.nn.sigmoid(row + b2v)
            o_ref[b, i:i + 1, :] = row * maskf[i:i + 1, :]


def kernel(structure, W1, b1, W2, b2):
    Bn, N, K = structure.shape
    H = b1.shape[0]
    b1c = b1.reshape(H, 1)
    b2c = b2.reshape(1, 1)
    out = pl.pallas_call(
        _mlp_kernel,
        grid=(1,),
        in_specs=[
            pl.BlockSpec((Bn, N, K), lambda t: (0, 0, 0)),
            pl.BlockSpec((2 * H, H), lambda t: (0, 0)),
            pl.BlockSpec((H, 1), lambda t: (0, 0)),
            pl.BlockSpec((H, 1), lambda t: (0, 0)),
            pl.BlockSpec((1, 1), lambda t: (0, 0)),
        ],
        out_specs=pl.BlockSpec((Bn, N, N), lambda t: (0, 0, 0)),
        out_shape=jax.ShapeDtypeStruct((Bn, N, N), jnp.float32),
    )(structure, W1, b1c, W2, b2c)
    return out


# speedup vs baseline: 1.0545x; 1.0545x over previous
import jax
import jax.numpy as jnp
from jax.experimental import pallas as pl

_G = 4


def _mlp_kernel(s_ref, w1_ref, b1_ref, w2b_ref, b2_ref, o_ref):
    H = b1_ref.shape[0]
    Bn, n, _ = s_ref.shape
    W1a = w1_ref[:H, :]
    W1b = w1_ref[H:, :]
    w2b = w2b_ref[...].astype(jnp.bfloat16)
    b2v = b2_ref[0, 0]
    zero = jnp.zeros((), jnp.bfloat16)

    for b in range(Bn):
        s = s_ref[b]
        AT = jax.lax.dot_general(W1a, s, (((0,), (1,)), ((), ())),
                                 preferred_element_type=jnp.float32) + b1_ref[...]
        BT = jax.lax.dot_general(W1b, s, (((0,), (1,)), ((), ())),
                                 preferred_element_type=jnp.float32)
        ATb = AT.astype(jnp.bfloat16)
        BTb = BT.astype(jnp.bfloat16)
        maskf = (s != 0).astype(jnp.float32)
        for i in range(0, n, _G):
            cols = ATb[:, i:i + _G]
            m = jnp.concatenate(
                [jnp.maximum(BTb + cols[:, g:g + 1], zero) for g in range(_G)],
                axis=0)
            rows = jax.lax.dot_general(w2b, m, (((1,), (0,)), ((), ())),
                                       preferred_element_type=jnp.float32)
            rows = jax.nn.sigmoid(rows + b2v)
            o_ref[b, i:i + _G, :] = rows * maskf[i:i + _G, :]


def kernel(structure, W1, b1, W2, b2):
    Bn, N, K = structure.shape
    H = b1.shape[0]
    b1c = b1.reshape(H, 1)
    b2c = b2.reshape(1, 1)
    w2blk = jnp.kron(jnp.eye(_G, dtype=W2.dtype), W2.reshape(1, H))
    out = pl.pallas_call(
        _mlp_kernel,
        grid=(1,),
        in_specs=[
            pl.BlockSpec((Bn, N, K), lambda t: (0, 0, 0)),
            pl.BlockSpec((2 * H, H), lambda t: (0, 0)),
            pl.BlockSpec((H, 1), lambda t: (0, 0)),
            pl.BlockSpec((_G, _G * H), lambda t: (0, 0)),
            pl.BlockSpec((1, 1), lambda t: (0, 0)),
        ],
        out_specs=pl.BlockSpec((Bn, N, N), lambda t: (0, 0, 0)),
        out_shape=jax.ShapeDtypeStruct((Bn, N, N), jnp.float32),
    )(structure, W1, b1c, w2blk, b2c)
    return out


# speedup vs baseline: 1.0646x; 1.0096x over previous
import jax
import jax.numpy as jnp
from jax.experimental import pallas as pl


def _mlp_kernel(s_ref, w1_ref, b1_ref, w2t_ref, b2_ref, o_ref):
    H = b1_ref.shape[0]
    Bn, n, _ = s_ref.shape
    W1a = w1_ref[:H, :].astype(jnp.bfloat16)
    W1b = w1_ref[H:, :].astype(jnp.bfloat16)
    w2t = w2t_ref[...].astype(jnp.bfloat16)
    b2v = b2_ref[0, 0]
    zero = jnp.zeros((), jnp.bfloat16)

    for b in range(Bn):
        s = s_ref[b]
        sb = s.astype(jnp.bfloat16)
        AT = jax.lax.dot_general(W1a, sb, (((0,), (1,)), ((), ())),
                                 preferred_element_type=jnp.float32) + b1_ref[...]
        BT = jax.lax.dot_general(W1b, sb, (((0,), (1,)), ((), ())),
                                 preferred_element_type=jnp.float32)
        ATb = AT.astype(jnp.bfloat16)
        BTb = BT.astype(jnp.bfloat16)
        maskf = (s != 0).astype(jnp.float32)
        for i in range(n):
            col = ATb[:, i:i + 1]
            m = jnp.maximum(BTb + col, zero)
            row = jax.lax.dot_general(w2t, m, (((1,), (0,)), ((), ())),
                                      preferred_element_type=jnp.float32)
            row = jax.nn.sigmoid(row + b2v)
            o_ref[b, i:i + 1, :] = row * maskf[i:i + 1, :]


def kernel(structure, W1, b1, W2, b2):
    Bn, N, K = structure.shape
    H = b1.shape[0]
    b1c = b1.reshape(H, 1)
    b2c = b2.reshape(1, 1)
    w2t = W2.reshape(1, H)
    out = pl.pallas_call(
        _mlp_kernel,
        grid=(1,),
        in_specs=[
            pl.BlockSpec((Bn, N, K), lambda t: (0, 0, 0)),
            pl.BlockSpec((2 * H, H), lambda t: (0, 0)),
            pl.BlockSpec((H, 1), lambda t: (0, 0)),
            pl.BlockSpec((1, H), lambda t: (0, 0)),
            pl.BlockSpec((1, 1), lambda t: (0, 0)),
        ],
        out_specs=pl.BlockSpec((Bn, N, N), lambda t: (0, 0, 0)),
        out_shape=jax.ShapeDtypeStruct((Bn, N, N), jnp.float32),
    )(structure, W1, b1c, w2t, b2c)
    return out
